# baseline (speedup 1.0000x reference)
import functools

import jax
import jax.numpy as jnp
from jax import lax
from jax.experimental import pallas as pl
from jax.experimental.pallas import tpu as pltpu
from jax.experimental.pallas import tpu_sc as plsc

_NUM_TOKENS = 100
_D_MODEL = 4096
_BATCH = 4
_NUM_CORES = 1
_NUM_SUBCORES = 16
_NUM_WORKERS = _NUM_CORES * _NUM_SUBCORES
_STRIPE = _D_MODEL // _NUM_WORKERS

_mesh = plsc.VectorSubcoreMesh(
    core_axis_name="c", subcore_axis_name="s", num_cores=_NUM_CORES
)


@functools.partial(
    pl.kernel,
    mesh=_mesh,
    out_type=jax.ShapeDtypeStruct((_BATCH, _NUM_TOKENS, _D_MODEL), jnp.float32),
    scratch_types=[
        pltpu.VMEM((_NUM_TOKENS, _STRIPE), jnp.float32),
        pltpu.SemaphoreType.DMA,
    ],
)
def _broadcast_kernel(table_hbm, out_hbm, buf, sem):
    wid = lax.axis_index("s") * _NUM_CORES + lax.axis_index("c")
    col = wid * _STRIPE
    pltpu.sync_copy(table_hbm.at[:, pl.ds(col, _STRIPE)], buf)
    copies = [
        pltpu.async_copy(buf, out_hbm.at[b].at[:, pl.ds(col, _STRIPE)], sem)
        for b in range(_BATCH)
    ]
    for c in copies:
        c.wait()


def kernel(batch_size, prompt_embeddings):
    del batch_size
    return _broadcast_kernel(prompt_embeddings)


# speedup vs baseline: 1.0279x; 1.0279x over previous
import functools

import jax
import jax.numpy as jnp
from jax import lax
from jax.experimental import pallas as pl
from jax.experimental.pallas import tpu as pltpu
from jax.experimental.pallas import tpu_sc as plsc

_NUM_TOKENS = 100
_D_MODEL = 4096
_BATCH = 4
_NUM_CORES = 2
_NUM_SUBCORES = 16
_NUM_WORKERS = _NUM_CORES * _NUM_SUBCORES
_STRIPE = _D_MODEL // _NUM_WORKERS

_mesh = plsc.VectorSubcoreMesh(
    core_axis_name="c", subcore_axis_name="s", num_cores=_NUM_CORES
)


@functools.partial(
    pl.kernel,
    mesh=_mesh,
    out_type=jax.ShapeDtypeStruct((_BATCH, _NUM_TOKENS, _D_MODEL), jnp.float32),
    scratch_types=[
        pltpu.VMEM((_NUM_TOKENS, _STRIPE), jnp.float32),
        pltpu.SemaphoreType.DMA,
    ],
)
def _broadcast_kernel(table_hbm, out_hbm, buf, sem):
    wid = lax.axis_index("s") * _NUM_CORES + lax.axis_index("c")
    col = wid * _STRIPE
    pltpu.sync_copy(table_hbm.at[:, pl.ds(col, _STRIPE)], buf)
    copies = [
        pltpu.async_copy(buf, out_hbm.at[b].at[:, pl.ds(col, _STRIPE)], sem)
        for b in range(_BATCH)
    ]
    for c in copies:
        c.wait()


def kernel(batch_size, prompt_embeddings):
    del batch_size
    return _broadcast_kernel(prompt_embeddings)
